# Initial kernel scaffold; baseline (speedup 1.0000x reference)
import functools

import jax
import jax.numpy as jnp
from jax import lax
from jax.experimental import pallas as pl
from jax.experimental.pallas import tpu as pltpu
from jax.experimental.pallas import tpu_sc as plsc

N_NODES = 100000
N_GRAPHS = 1024
D = 128
DJ = D // 16

NC = 2
NS = 16
NW = NC * NS
GPW = N_GRAPHS // NW

B = 200
SC_CH = 2000


def _body(x_hbm, batch_hbm, out_hbm, xbuf, bbuf, sbuf, sum_tab, max_tab,
          cnt_tab, stage):
    wid = lax.axis_index("s") * NC + lax.axis_index("c")
    g0 = wid * GPW

    zf = jnp.zeros((16,), jnp.float32)
    ninf = jnp.full((16,), -3.4028235e38, jnp.float32)

    def init_g(g, c):
        for j in range(DJ):
            sum_tab[g, pl.ds(j * 16, 16)] = zf
            max_tab[g, pl.ds(j * 16, 16)] = ninf
        cnt_tab[g] = zf
        return c

    lax.fori_loop(0, GPW, init_g, 0)

    g0v = jnp.full((16,), g0, jnp.int32)
    g1v = g0v + GPW
    zi = jnp.zeros((16,), jnp.int32)

    def scan_chunk(c, carry):
        pltpu.sync_copy(batch_hbm.at[pl.ds(c * SC_CH, SC_CH)], sbuf)

        def scan_vec(i, carry2):
            lo_c, hi_c = carry2
            v = sbuf[pl.ds(i * 16, 16)]
            lo_c = lo_c + plsc.all_reduce_population_count(v < g0v)
            hi_c = hi_c + plsc.all_reduce_population_count(v < g1v)
            return (lo_c, hi_c)

        return lax.fori_loop(0, SC_CH // 16, scan_vec, carry)

    lo_v, hi_v = lax.fori_loop(0, N_NODES // SC_CH, scan_chunk, (zi, zi))
    lo = jnp.max(lo_v)
    hi = jnp.max(hi_v)

    blk0 = (lo // B) * B
    nblk = (hi - blk0 + (B - 1)) // B

    def do_block(b, c):
        blk = blk0 + b * B
        pltpu.sync_copy(x_hbm.at[pl.ds(blk, B)], xbuf)
        pltpu.sync_copy(batch_hbm.at[pl.ds(blk, B)], bbuf)
        r0 = jnp.maximum(lo - blk, 0)
        r1 = jnp.minimum(hi - blk, B)

        def do_row(r, c2):
            bl = bbuf[r] - g0
            for j in range(DJ):
                row = xbuf[r, pl.ds(j * 16, 16)]
                s = sum_tab[bl, pl.ds(j * 16, 16)]
                sum_tab[bl, pl.ds(j * 16, 16)] = s + row
                m = max_tab[bl, pl.ds(j * 16, 16)]
                max_tab[bl, pl.ds(j * 16, 16)] = jnp.maximum(m, row)
            cnt_tab[bl] = cnt_tab[bl] + 1.0
            return c2

        lax.fori_loop(r0, r1, do_row, 0)
        return c

    lax.fori_loop(0, nblk, do_block, 0)

    def fin(g, c):
        cb = cnt_tab[g]
        cpos = cb > 0.0
        denom = jnp.maximum(cb, 1.0)
        for j in range(DJ):
            s = sum_tab[g, pl.ds(j * 16, 16)]
            m = max_tab[g, pl.ds(j * 16, 16)]
            stage[g, pl.ds(j * 16, 16)] = s / denom
            stage[g, pl.ds(D + j * 16, 16)] = jnp.where(cpos, m, 0.0)
            stage[g, pl.ds(2 * D + j * 16, 16)] = s
        return c

    lax.fori_loop(0, GPW, fin, 0)
    pltpu.sync_copy(stage, out_hbm.at[pl.ds(g0, GPW)])


@jax.jit
def _pool(x, batch):
    mesh = plsc.VectorSubcoreMesh(core_axis_name="c", subcore_axis_name="s")
    run = pl.kernel(
        _body,
        out_type=jax.ShapeDtypeStruct((N_GRAPHS, 3 * D), jnp.float32),
        mesh=mesh,
        scratch_types=[
            pltpu.VMEM((B, D), jnp.float32),
            pltpu.VMEM((B,), jnp.int32),
            pltpu.VMEM((SC_CH,), jnp.int32),
            pltpu.VMEM((GPW, D), jnp.float32),
            pltpu.VMEM((GPW, D), jnp.float32),
            pltpu.VMEM((GPW, 16), jnp.float32),
            pltpu.VMEM((GPW, 3 * D), jnp.float32),
        ],
    )
    return run(x, batch)


def kernel(x, batch):
    return _pool(x, batch.astype(jnp.int32))


# speedup vs baseline: 2.6985x; 2.6985x over previous
import functools

import jax
import jax.numpy as jnp
from jax import lax
from jax.experimental import pallas as pl
from jax.experimental.pallas import tpu as pltpu
from jax.experimental.pallas import tpu_sc as plsc

N_NODES = 100000
N_GRAPHS = 1024
D = 128
DJ = D // 16

NC = 2
NS = 16
NW = NC * NS
GPW = N_GRAPHS // NW

B = 200
SC_CH = 2000


def _worker_id():
    return lax.axis_index("s") * NC + lax.axis_index("c")


def _body(x_hbm, batch_hbm, out_hbm, xbuf, bbuf, sbuf, sum_tab, max_tab,
          cnt_tab, stage):
    wid = _worker_id()
    g0 = wid * GPW

    zf = jnp.zeros((16,), jnp.float32)
    ninf = jnp.full((16,), -3.4028235e38, jnp.float32)

    def init_g(g, c):
        for j in range(DJ):
            sum_tab[g, pl.ds(j * 16, 16)] = zf
            max_tab[g, pl.ds(j * 16, 16)] = ninf
        cnt_tab[g] = zf
        return c

    lax.fori_loop(0, GPW, init_g, 0)

    g0v = jnp.full((16,), g0, jnp.int32)
    g1v = g0v + GPW

    def scan_chunk(c, carry):
        pltpu.sync_copy(batch_hbm.at[pl.ds(c * SC_CH, SC_CH)], sbuf)

        def scan_vec(i, carry2):
            lo_c, hi_c = carry2
            v = sbuf[pl.ds(i * 16, 16)]
            lo_c = lo_c + jnp.sum((v < g0v).astype(jnp.int32))
            hi_c = hi_c + jnp.sum((v < g1v).astype(jnp.int32))
            return (lo_c, hi_c)

        return lax.fori_loop(0, SC_CH // 16, scan_vec, carry)

    lo, hi = lax.fori_loop(0, N_NODES // SC_CH, scan_chunk,
                           (jnp.int32(0), jnp.int32(0)))

    blk0 = (lo // B) * B
    nblk = (hi - blk0 + (B - 1)) // B

    def do_block(b, c):
        blk = blk0 + b * B
        pltpu.sync_copy(x_hbm.at[pl.ds(blk, B)], xbuf)
        pltpu.sync_copy(batch_hbm.at[pl.ds(blk, B)], bbuf.at[pl.ds(0, B)])
        r0 = jnp.maximum(lo - blk, 0)
        r1 = jnp.minimum(hi - blk, B)

        def do_row(r, c2):
            bl = bbuf[pl.ds(r, 16)][0] - g0
            for j in range(DJ):
                row = xbuf[r, pl.ds(j * 16, 16)]
                s = sum_tab[bl, pl.ds(j * 16, 16)]
                sum_tab[bl, pl.ds(j * 16, 16)] = s + row
                m = max_tab[bl, pl.ds(j * 16, 16)]
                max_tab[bl, pl.ds(j * 16, 16)] = jnp.maximum(m, row)
            cnt_tab[bl] = cnt_tab[bl] + 1.0
            return c2

        lax.fori_loop(r0, r1, do_row, 0)
        return c

    lax.fori_loop(0, nblk, do_block, 0)

    def fin(g, c):
        cb = cnt_tab[g]
        cpos = cb > 0.0
        denom = jnp.maximum(cb, 1.0)
        for j in range(DJ):
            s = sum_tab[g, pl.ds(j * 16, 16)]
            m = max_tab[g, pl.ds(j * 16, 16)]
            stage[g, pl.ds(j * 16, 16)] = s / denom
            stage[g, pl.ds(D + j * 16, 16)] = jnp.where(cpos, m, 0.0)
            stage[g, pl.ds(2 * D + j * 16, 16)] = s
        return c

    lax.fori_loop(0, GPW, fin, 0)
    pltpu.sync_copy(stage, out_hbm.at[pl.ds(g0, GPW)])


@jax.jit
def _pool(x, batch):
    mesh = plsc.VectorSubcoreMesh(core_axis_name="c", subcore_axis_name="s",
                                  num_cores=NC, num_subcores=NS)
    run = pl.kernel(
        _body,
        out_type=jax.ShapeDtypeStruct((N_GRAPHS, 3 * D), jnp.float32),
        mesh=mesh,
        compiler_params=pltpu.CompilerParams(needs_layout_passes=False),
        scratch_types=[
            pltpu.VMEM((B, D), jnp.float32),
            pltpu.VMEM((B + 16,), jnp.int32),
            pltpu.VMEM((SC_CH,), jnp.int32),
            pltpu.VMEM((GPW, D), jnp.float32),
            pltpu.VMEM((GPW, D), jnp.float32),
            pltpu.VMEM((GPW, 16), jnp.float32),
            pltpu.VMEM((GPW, 3 * D), jnp.float32),
        ],
    )
    return run(x, batch)


def kernel(x, batch):
    return _pool(x, batch.astype(jnp.int32))


# speedup vs baseline: 8.4495x; 3.1311x over previous
import functools

import jax
import jax.numpy as jnp
from jax import lax
from jax.experimental import pallas as pl
from jax.experimental.pallas import tpu as pltpu
from jax.experimental.pallas import tpu_sc as plsc

N_NODES = 100000
N_GRAPHS = 1024
D = 128
DJ = D // 16

NC = 2
NS = 16
NW = NC * NS
GPW = N_GRAPHS // NW

B = 200
SC_CH = 10000

NEG_INF = -3.4028235e38


def _worker_id():
    return lax.axis_index("s") * NC + lax.axis_index("c")


def _popcnt(mask):
    return plsc.all_reduce_population_count(mask)


def _ffs(mask):
    return plsc.all_reduce_ffs(mask)


def _body(x_hbm, batch_hbm, out_hbm, xb0, xb1, bb0, bb1, sbuf, stage,
          semx0, semx1):
    wid = _worker_id()
    g0 = wid * GPW

    zf = jnp.zeros((16,), jnp.float32)

    def init_g(g, c):
        for j in range(3 * DJ):
            stage[g, pl.ds(j * 16, 16)] = zf
        return c

    lax.fori_loop(0, GPW, init_g, 0)

    g0v = jnp.full((16,), g0, jnp.int32)
    g1v = g0v + GPW
    zi = jnp.zeros((16,), jnp.int32)

    def scan_chunk(c, carry):
        pltpu.sync_copy(batch_hbm.at[pl.ds(c * SC_CH, SC_CH)], sbuf)

        def scan_vec(i, carry2):
            lo_c, hi_c = carry2
            v = sbuf[pl.ds(i * 16, 16)]
            lo_c = lo_c + _popcnt(v < g0v)
            hi_c = hi_c + _popcnt(v < g1v)
            return (lo_c, hi_c)

        return lax.fori_loop(0, SC_CH // 16, scan_vec, carry)

    lo_v, hi_v = lax.fori_loop(0, N_NODES // SC_CH, scan_chunk, (zi, zi))
    lo = lo_v[0]
    hi = hi_v[0]

    blk0 = (lo // B) * B
    nblk = (hi - blk0 + (B - 1)) // B

    def issue(blk, xb, bb, sem):
        pltpu.make_async_copy(x_hbm.at[pl.ds(blk, B)], xb, sem).start()
        pltpu.make_async_copy(batch_hbm.at[pl.ds(blk, B)],
                              bb.at[pl.ds(0, B)], sem).start()

    def drain(xb, bb, sem):
        pltpu.make_async_copy(x_hbm.at[pl.ds(0, B)], xb, sem).wait()
        pltpu.make_async_copy(batch_hbm.at[pl.ds(0, B)],
                              bb.at[pl.ds(0, B)], sem).wait()

    def flush(prev_id, cnt_s, s, m):
        bl = prev_id - g0
        cntf = jnp.full((16,), cnt_s).astype(jnp.float32)
        inv = 1.0 / cntf
        for j in range(DJ):
            stage[bl, pl.ds(j * 16, 16)] = s[j] * inv
            stage[bl, pl.ds(D + j * 16, 16)] = m[j]
            stage[bl, pl.ds(2 * D + j * 16, 16)] = s[j]

    lanes = lax.iota(jnp.int32, 16)

    def process(blk, xb, bb, carry):
        r0 = jnp.minimum(jnp.maximum(lo - blk, 0), B)
        r1 = jnp.minimum(jnp.maximum(hi - blk, 0), B)

        def run_cond(st):
            return st[0] < r1

        def run_body(st):
            r, prev_id, cnt_s, s, m = st
            cur = bb[pl.ds(r, 16)][0]
            changed = cur != prev_id

            @pl.when(changed & (cnt_s > 0))
            def _():
                flush(prev_id, cnt_s, s, m)

            s = tuple(jnp.where(changed, zf, sj) for sj in s)
            m = tuple(jnp.where(changed, NEG_INF, mj) for mj in m)
            cnt_s = jnp.where(changed, 0, cnt_s)

            cur_v = jnp.full((16,), cur, jnp.int32)

            def se_cond(st2):
                return (st2[1] < 0) & (st2[0] < r1)

            def se_body(st2):
                rr, _e = st2
                chunk = bb[pl.ds(rr, 16)]
                mm = (chunk != cur_v) | ((lanes + rr) >= r1)
                pc = _popcnt(mm)
                fi = _ffs(mm)
                e_v = jnp.where(pc > 0, rr + fi, -1)
                return (rr + 16, e_v[0])

            _rr, e = lax.while_loop(se_cond, se_body, (r, jnp.int32(-1)))
            e = jnp.where(e < 0, r1, e)

            def acc(rr, sm):
                s2, m2 = sm
                s3, m3 = [], []
                for j in range(DJ):
                    v = xb[rr, pl.ds(j * 16, 16)]
                    s3.append(s2[j] + v)
                    m3.append(jnp.maximum(m2[j], v))
                return (tuple(s3), tuple(m3))

            s, m = lax.fori_loop(r, e, acc, (s, m))
            cnt_s = cnt_s + (e - r)
            return (e, cur, cnt_s, s, m)

        r, prev_id, cnt_s, s, m = lax.while_loop(
            run_cond, run_body, (r0,) + carry)
        return (prev_id, cnt_s, s, m)

    carry0 = (jnp.int32(-1), jnp.int32(0),
              tuple(zf for _ in range(DJ)),
              tuple(jnp.full((16,), NEG_INF, jnp.float32) for _ in range(DJ)))

    @pl.when(nblk > 0)
    def _():
        issue(blk0, xb0, bb0, semx0)

    npair = (nblk + 1) // 2

    def do_pair(g, carry):
        b0 = 2 * g
        b1 = b0 + 1
        blk_a = blk0 + b0 * B
        blk_b = blk_a + B

        @pl.when(b1 < nblk)
        def _():
            issue(blk_b, xb1, bb1, semx1)

        drain(xb0, bb0, semx0)
        carry = process(blk_a, xb0, bb0, carry)

        @pl.when(b1 + 1 < nblk)
        def _():
            issue(blk_b + B, xb0, bb0, semx0)

        @pl.when(b1 < nblk)
        def _():
            drain(xb1, bb1, semx1)

        carry = process(blk_b, xb1, bb1, carry)
        return carry

    prev_id, cnt_s, s, m = lax.fori_loop(0, npair, do_pair, carry0)

    @pl.when(cnt_s > 0)
    def _():
        flush(prev_id, cnt_s, s, m)

    pltpu.sync_copy(stage, out_hbm.at[pl.ds(g0, GPW)])


@jax.jit
def _pool(x, batch):
    mesh = plsc.VectorSubcoreMesh(core_axis_name="c", subcore_axis_name="s",
                                  num_cores=NC, num_subcores=NS)
    run = pl.kernel(
        _body,
        out_type=jax.ShapeDtypeStruct((N_GRAPHS, 3 * D), jnp.float32),
        mesh=mesh,
        compiler_params=pltpu.CompilerParams(needs_layout_passes=False),
        scratch_types=[
            pltpu.VMEM((B, D), jnp.float32),
            pltpu.VMEM((B, D), jnp.float32),
            pltpu.VMEM((B + 16,), jnp.int32),
            pltpu.VMEM((B + 16,), jnp.int32),
            pltpu.VMEM((SC_CH,), jnp.int32),
            pltpu.VMEM((GPW, 3 * D), jnp.float32),
            pltpu.SemaphoreType.DMA,
            pltpu.SemaphoreType.DMA,
        ],
    )
    return run(x, batch)


def kernel(x, batch):
    return _pool(x, batch.astype(jnp.int32))
